# Initial kernel scaffold; baseline (speedup 1.0000x reference)
import jax
import jax.numpy as jnp
from jax.experimental import pallas as pl


def kernel(input, embed_weight):
    raise NotImplementedError("write your pallas kernel here")


# speedup vs baseline: 1.4758x; 1.4758x over previous
import functools

import jax
import jax.numpy as jnp
from jax import lax
from jax.experimental import pallas as pl
from jax.experimental.pallas import tpu as pltpu
from jax.experimental.pallas import tpu_sc as plsc

NC = 2
NS = 16
NW = NC * NS

CHUNK = 8


@functools.partial(jax.jit, static_argnums=(2, 3))
def _gather_rows(table, idx, n, d):
    b_per_w = n // NW
    n_chunks = b_per_w // CHUNK

    mesh = plsc.VectorSubcoreMesh(core_axis_name="c", subcore_axis_name="s")

    @functools.partial(
        pl.kernel,
        mesh=mesh,
        out_type=jax.ShapeDtypeStruct((n, d), jnp.float32),
        scratch_types=[
            pltpu.VMEM((b_per_w,), jnp.int32),
            pltpu.VMEM((CHUNK, d), jnp.float32),
            pltpu.SemaphoreType.DMA,
        ],
    )
    def k(table_hbm, idx_hbm, out_hbm, idx_v, buf, sem):
        wid = lax.axis_index("s") * NC + lax.axis_index("c")
        base = wid * b_per_w
        pltpu.sync_copy(idx_hbm.at[pl.ds(base, b_per_w)], idx_v)

        def chunk_body(g, carry):
            off = pl.multiple_of(g * CHUNK, CHUNK)
            pltpu.async_copy(
                table_hbm.at[idx_v.at[pl.ds(off, CHUNK)]], buf, sem
            ).wait()
            pltpu.sync_copy(buf, out_hbm.at[pl.ds(base + off, CHUNK)])
            return carry

        lax.fori_loop(0, n_chunks, chunk_body, 0)

    return k(table, idx)


def kernel(input, embed_weight):
    b, t = input.shape
    v, d = embed_weight.shape
    idx = input.reshape(b * t).astype(jnp.int32)
    out = _gather_rows(embed_weight, idx, b * t, d)
    return out.reshape(b, t, d)


# speedup vs baseline: 1.7293x; 1.1718x over previous
import functools

import jax
import jax.numpy as jnp
from jax import lax
from jax.experimental import pallas as pl
from jax.experimental.pallas import tpu as pltpu
from jax.experimental.pallas import tpu_sc as plsc

NC = 2
NS = 16
NW = NC * NS

CHUNK = 8


@functools.partial(jax.jit, static_argnums=(2, 3))
def _gather_rows(table, idx, n, d):
    b_per_w = n // NW
    n_chunks = b_per_w // CHUNK

    mesh = plsc.VectorSubcoreMesh(core_axis_name="c", subcore_axis_name="s")

    @functools.partial(
        pl.kernel,
        mesh=mesh,
        out_type=jax.ShapeDtypeStruct((n, d), jnp.float32),
        scratch_types=[
            pltpu.VMEM((b_per_w,), jnp.int32),
            pltpu.VMEM((CHUNK, d), jnp.float32),
            pltpu.VMEM((CHUNK, d), jnp.float32),
            pltpu.SemaphoreType.DMA,
            pltpu.SemaphoreType.DMA,
            pltpu.SemaphoreType.DMA,
            pltpu.SemaphoreType.DMA,
        ],
    )
    def k(table_hbm, idx_hbm, out_hbm, idx_v, buf0, buf1,
          gsem0, gsem1, ssem0, ssem1):
        wid = lax.axis_index("s") * NC + lax.axis_index("c")
        base = wid * b_per_w
        pltpu.sync_copy(idx_hbm.at[pl.ds(base, b_per_w)], idx_v)

        bufs = (buf0, buf1)
        gsems = (gsem0, gsem1)
        ssems = (ssem0, ssem1)

        def start_gather(g, b):
            off = pl.multiple_of(g * CHUNK, CHUNK)
            return pltpu.async_copy(
                table_hbm.at[idx_v.at[pl.ds(off, CHUNK)]], bufs[b], gsems[b]
            )

        def start_scatter(g, b):
            off = pl.multiple_of(g * CHUNK, CHUNK)
            return pltpu.async_copy(
                bufs[b], out_hbm.at[pl.ds(base + off, CHUNK)], ssems[b]
            )

        def wait_scatter(b):
            pltpu.make_async_copy(
                bufs[b], out_hbm.at[pl.ds(base, CHUNK)], ssems[b]
            ).wait()

        gd0 = start_gather(0, 0)
        gd1 = start_gather(1, 1)
        gd0.wait()
        start_scatter(0, 0)
        gd1.wait()
        start_scatter(1, 1)

        def pair_body(p, carry):
            for b in range(2):
                g = p * 2 + b
                wait_scatter(b)
                gd = start_gather(g, b)
                gd.wait()
                start_scatter(g, b)
            return carry

        lax.fori_loop(1, n_chunks // 2, pair_body, 0)

        for b in range(2):
            wait_scatter(b)

    return k(table, idx)


def kernel(input, embed_weight):
    b, t = input.shape
    v, d = embed_weight.shape
    idx = input.reshape(b * t).astype(jnp.int32)
    out = _gather_rows(embed_weight, idx, b * t, d)
    return out.reshape(b, t, d)


# speedup vs baseline: 1.7656x; 1.0210x over previous
import functools

import jax
import jax.numpy as jnp
from jax import lax
from jax.experimental import pallas as pl
from jax.experimental.pallas import tpu as pltpu
from jax.experimental.pallas import tpu_sc as plsc

NC = 2
NS = 16
NW = NC * NS

CHUNK = 8


@functools.partial(jax.jit, static_argnums=(2, 3))
def _gather_rows(table, idx, n, d):
    b_per_w = n // NW
    n_chunks = b_per_w // CHUNK

    mesh = plsc.VectorSubcoreMesh(core_axis_name="c", subcore_axis_name="s")

    @functools.partial(
        pl.kernel,
        mesh=mesh,
        out_type=jax.ShapeDtypeStruct((n, d), jnp.float32),
        scratch_types=[
            pltpu.VMEM((b_per_w,), jnp.int32),
            pltpu.VMEM((CHUNK, d), jnp.float32),
            pltpu.VMEM((CHUNK, d), jnp.float32),
            pltpu.SemaphoreType.DMA,
            pltpu.SemaphoreType.DMA,
            pltpu.SemaphoreType.DMA,
            pltpu.SemaphoreType.DMA,
        ],
    )
    def k(table_hbm, idx_hbm, out_hbm, idx_v, buf0, buf1,
          gsem0, gsem1, ssem0, ssem1):
        wid = lax.axis_index("s") * NC + lax.axis_index("c")
        base = wid * b_per_w
        pltpu.sync_copy(idx_hbm.at[pl.ds(base, b_per_w)], idx_v)

        bufs = (buf0, buf1)
        gsems = (gsem0, gsem1)
        ssems = (ssem0, ssem1)

        def start_gather(g, b):
            off = pl.multiple_of(g * CHUNK, CHUNK)
            return pltpu.async_copy(
                table_hbm.at[idx_v.at[pl.ds(off, CHUNK)]], bufs[b], gsems[b]
            )

        def start_scatter(g, b):
            off = pl.multiple_of(g * CHUNK, CHUNK)
            return pltpu.async_copy(
                bufs[b], out_hbm.at[pl.ds(base + off, CHUNK)], ssems[b]
            )

        def wait_scatter(b):
            pltpu.make_async_copy(
                bufs[b], out_hbm.at[pl.ds(base, CHUNK)], ssems[b]
            ).wait()

        def wait_gather(b):
            pltpu.make_async_copy(
                table_hbm.at[idx_v.at[pl.ds(0, CHUNK)]], bufs[b], gsems[b]
            ).wait()

        start_gather(0, 0)
        start_gather(1, 1)
        wait_gather(0)
        start_scatter(0, 0)

        def pair_body(p, carry):
            for b in range(2):
                g = p * 2 + b
                wait_scatter(b)
                start_gather(g, b)
                wait_gather(1 - b)
                start_scatter(g - 1, 1 - b)
            return carry

        lax.fori_loop(1, n_chunks // 2, pair_body, 0)

        wait_gather(1)
        start_scatter(n_chunks - 1, 1)
        wait_scatter(0)
        wait_scatter(1)

    return k(table, idx)


def kernel(input, embed_weight):
    b, t = input.shape
    v, d = embed_weight.shape
    idx = input.reshape(b * t).astype(jnp.int32)
    out = _gather_rows(embed_weight, idx, b * t, d)
    return out.reshape(b, t, d)
